# baseline (speedup 1.0000x reference)
import jax
import jax.numpy as jnp
from jax import lax
from jax.experimental import pallas as pl
from jax.experimental.pallas import tpu as pltpu
from jax.experimental.pallas import tpu_sc as plsc

VOCAB = 1000000
D = 64
B = 16384 * 50

NC = 2
NS = 16
NW = NC * NS
B_PER_W = B // NW
IDX_ROW = 128
N_ROWS = B_PER_W // IDX_ROW
GB = 4
TB = 2


def _row_coords(r_global):
    t = lax.shift_right_logical(r_global, 7)
    tj = lax.bitwise_and(r_global, 127)
    return t, tj


def _embed_body(x_hbm, table_hbm, out_hbm, idx_v,
                g0, g1, g2, g3, t0, t1,
                gs0, gs1, gs2, gs3, os0, os1):
    wid = lax.axis_index("s") * NC + lax.axis_index("c")
    pltpu.sync_copy(x_hbm.at[wid], idx_v)
    row_base = wid * N_ROWS
    gbufs = (g0, g1, g2, g3)
    tbufs = (t0, t1)
    gsems = (gs0, gs1, gs2, gs3)
    osems = (os0, os1)
    iota16 = lax.iota(jnp.int32, 16)

    def fire_g(r, slot):
        pltpu.async_copy(table_hbm.at[idx_v.at[r]], gbufs[slot], gsems[slot])

    def drain_g(r, slot):
        pltpu.make_async_copy(
            table_hbm.at[idx_v.at[r]], gbufs[slot], gsems[slot]).wait()

    def fire_outs(r, slot):
        t, tj = _row_coords(row_base + r)
        for ti in range(8):
            pltpu.async_copy(
                tbufs[slot].at[pl.ds(ti * 1024, 1024)],
                out_hbm.at[t, ti, tj], osems[slot])

    def wait_outs(r, slot):
        t, tj = _row_coords(row_base + r)
        for ti in range(8):
            pltpu.make_async_copy(
                tbufs[slot].at[pl.ds(ti * 1024, 1024)],
                out_hbm.at[t, ti, tj], osems[slot]).wait()

    bvecs = [iota16 + (c * 16) for c in range(8)]

    def transpose(gslot, tslot):
        gb, tb = gbufs[gslot], tbufs[tslot]

        @pl.loop(0, D, step=8)
        def _d(d0):
            for dj in range(8):
                d = d0 + dj
                dvec = jnp.full((16,), d, jnp.int32)
                base = d * 128
                for c in range(8):
                    vals = plsc.load_gather(gb, [bvecs[c], dvec])
                    tb[pl.ds(base + c * 16, 16)] = vals

    for r in range(GB):
        fire_g(r, r)

    @pl.loop(0, N_ROWS, step=GB)
    def _rows(r0):
        for j in range(GB):
            r = r0 + j
            gslot, tslot = j % GB, j % TB
            drain_g(r, gslot)
            if j < TB:
                pl.when(r0 > 0)(lambda rr=r, ts=tslot: wait_outs(rr - TB, ts))
            else:
                wait_outs(r - TB, tslot)
            transpose(gslot, tslot)
            fire_outs(r, tslot)
            pl.when(r0 < N_ROWS - GB)(
                lambda rr=r, gs=gslot: fire_g(rr + GB, gs))

    wait_outs(N_ROWS - 2, 0)
    wait_outs(N_ROWS - 1, 1)


@jax.jit
def _embed(x_flat3, table):
    mesh = plsc.VectorSubcoreMesh(core_axis_name="c", subcore_axis_name="s")
    return pl.kernel(
        _embed_body,
        out_type=jax.ShapeDtypeStruct((50, 8, 128, 1024), jnp.float32),
        mesh=mesh,
        compiler_params=pltpu.CompilerParams(
            use_tc_tiling_on_sc=False, needs_layout_passes=False),
        scratch_types=[
            pltpu.VMEM((N_ROWS, IDX_ROW), jnp.int32),
            pltpu.VMEM((IDX_ROW, D), jnp.float32),
            pltpu.VMEM((IDX_ROW, D), jnp.float32),
            pltpu.VMEM((IDX_ROW, D), jnp.float32),
            pltpu.VMEM((IDX_ROW, D), jnp.float32),
            pltpu.VMEM((8192,), jnp.float32),
            pltpu.VMEM((8192,), jnp.float32),
            pltpu.SemaphoreType.DMA,
            pltpu.SemaphoreType.DMA,
            pltpu.SemaphoreType.DMA,
            pltpu.SemaphoreType.DMA,
            pltpu.SemaphoreType.DMA,
            pltpu.SemaphoreType.DMA,
        ],
    )(x_flat3, table)


def kernel(x, table):
    x_flat3 = x.T.reshape(NW, N_ROWS, IDX_ROW).astype(jnp.int32)
    out5 = _embed(x_flat3, table).reshape(50, 8, 128, 8, 128)
    return out5.transpose(2, 4, 0, 1, 3).reshape(16384, 50, D)


# speedup vs baseline: 1.3825x; 1.3825x over previous
import jax
import jax.numpy as jnp
from jax import lax
from jax.experimental import pallas as pl
from jax.experimental.pallas import tpu as pltpu
from jax.experimental.pallas import tpu_sc as plsc

VOCAB = 1000000
D = 64
B = 16384 * 50

NC = 2
NS = 16
NW = NC * NS
B_PER_W = B // NW
IDX_ROW = 128
N_ROWS = B_PER_W // IDX_ROW
GB = 4
TB = 2


def _row_coords(r_global):
    t = lax.shift_right_logical(r_global, 7)
    tj = lax.bitwise_and(r_global, 127)
    return t, tj


def _embed_body(x_hbm, table_hbm, out_hbm, idx_v,
                g0, g1, g2, g3, t0, t1,
                gs0, gs1, gs2, gs3, os0, os1):
    wid = lax.axis_index("s") * NC + lax.axis_index("c")
    pltpu.sync_copy(x_hbm.at[wid], idx_v)
    row_base = wid * N_ROWS
    gbufs = (g0, g1, g2, g3)
    tbufs = (t0, t1)
    gsems = (gs0, gs1, gs2, gs3)
    osems = (os0, os1)
    iota16 = lax.iota(jnp.int32, 16)

    def fire_g(r, slot):
        pltpu.async_copy(table_hbm.at[idx_v.at[r]], gbufs[slot], gsems[slot])

    def drain_g(r, slot):
        pltpu.make_async_copy(
            table_hbm.at[idx_v.at[r]], gbufs[slot], gsems[slot]).wait()

    def fire_outs(r, slot):
        t, tj = _row_coords(row_base + r)
        for ti in range(8):
            pltpu.async_copy(
                tbufs[slot].at[pl.ds(ti * 1024, 1024)],
                out_hbm.at[t, ti, tj], osems[slot])

    def wait_outs(r, slot):
        t, tj = _row_coords(row_base + r)
        for ti in range(8):
            pltpu.make_async_copy(
                tbufs[slot].at[pl.ds(ti * 1024, 1024)],
                out_hbm.at[t, ti, tj], osems[slot]).wait()

    bvecs = [iota16 + (c * 16) for c in range(8)]

    def transpose(gslot, tslot):
        gb, tb = gbufs[gslot], tbufs[tslot]

        @plsc.parallel_loop(0, D, step=8)
        def _d(d0):
            for dj in range(8):
                d = d0 + dj
                dvec = jnp.full((16,), d, jnp.int32)
                base = d * 128
                for c in range(8):
                    vals = plsc.load_gather(gb, [bvecs[c], dvec])
                    tb[pl.ds(base + c * 16, 16)] = vals

    for r in range(GB):
        fire_g(r, r)

    @pl.loop(0, N_ROWS, step=GB)
    def _rows(r0):
        for j in range(GB):
            r = r0 + j
            gslot, tslot = j % GB, j % TB
            drain_g(r, gslot)
            if j < TB:
                pl.when(r0 > 0)(lambda rr=r, ts=tslot: wait_outs(rr - TB, ts))
            else:
                wait_outs(r - TB, tslot)
            transpose(gslot, tslot)
            fire_outs(r, tslot)
            pl.when(r0 < N_ROWS - GB)(
                lambda rr=r, gs=gslot: fire_g(rr + GB, gs))

    wait_outs(N_ROWS - 2, 0)
    wait_outs(N_ROWS - 1, 1)


@jax.jit
def _embed(x_flat3, table):
    mesh = plsc.VectorSubcoreMesh(core_axis_name="c", subcore_axis_name="s")
    return pl.kernel(
        _embed_body,
        out_type=jax.ShapeDtypeStruct((50, 8, 128, 1024), jnp.float32),
        mesh=mesh,
        compiler_params=pltpu.CompilerParams(
            use_tc_tiling_on_sc=False, needs_layout_passes=False),
        scratch_types=[
            pltpu.VMEM((N_ROWS, IDX_ROW), jnp.int32),
            pltpu.VMEM((IDX_ROW, D), jnp.float32),
            pltpu.VMEM((IDX_ROW, D), jnp.float32),
            pltpu.VMEM((IDX_ROW, D), jnp.float32),
            pltpu.VMEM((IDX_ROW, D), jnp.float32),
            pltpu.VMEM((8192,), jnp.float32),
            pltpu.VMEM((8192,), jnp.float32),
            pltpu.SemaphoreType.DMA,
            pltpu.SemaphoreType.DMA,
            pltpu.SemaphoreType.DMA,
            pltpu.SemaphoreType.DMA,
            pltpu.SemaphoreType.DMA,
            pltpu.SemaphoreType.DMA,
        ],
    )(x_flat3, table)


def kernel(x, table):
    x_flat3 = x.T.reshape(NW, N_ROWS, IDX_ROW).astype(jnp.int32)
    out5 = _embed(x_flat3, table).reshape(50, 8, 128, 8, 128)
    return out5.transpose(2, 4, 0, 1, 3).reshape(16384, 50, D)


# speedup vs baseline: 1.4425x; 1.0434x over previous
import jax
import jax.numpy as jnp
from jax import lax
from jax.experimental import pallas as pl
from jax.experimental.pallas import tpu as pltpu
from jax.experimental.pallas import tpu_sc as plsc

VOCAB = 1000000
D = 64
B = 16384 * 50

NC = 2
NS = 16
NW = NC * NS
B_PER_W = B // NW
IDX_ROW = 128
N_ROWS = B_PER_W // IDX_ROW
GB = 4
TB = 2


def _row_coords(r_global):
    t = lax.shift_right_logical(r_global, 7)
    tj = lax.bitwise_and(r_global, 127)
    return t, tj


def _embed_body(x_hbm, table_hbm, out_hbm, idx_v,
                g0, g1, g2, g3, t0, t1,
                gs0, gs1, gs2, gs3, os0, os1):
    wid = lax.axis_index("s") * NC + lax.axis_index("c")
    pltpu.sync_copy(x_hbm.at[wid], idx_v)
    row_base = wid * N_ROWS
    gbufs = (g0, g1, g2, g3)
    tbufs = (t0, t1)
    gsems = (gs0, gs1, gs2, gs3)
    osems = (os0, os1)
    iota16 = lax.iota(jnp.int32, 16)

    def fire_g(r, slot):
        pltpu.async_copy(table_hbm.at[idx_v.at[r]], gbufs[slot], gsems[slot])

    def drain_g(r, slot):
        pltpu.make_async_copy(
            table_hbm.at[idx_v.at[r]], gbufs[slot], gsems[slot]).wait()

    def fire_outs(r, slot):
        t, tj = _row_coords(row_base + r)
        for ti in range(8):
            pltpu.async_copy(
                tbufs[slot].at[pl.ds(ti * 1024, 1024)],
                out_hbm.at[t, ti, tj], osems[slot])

    def wait_outs(r, slot):
        t, tj = _row_coords(row_base + r)
        for ti in range(8):
            pltpu.make_async_copy(
                tbufs[slot].at[pl.ds(ti * 1024, 1024)],
                out_hbm.at[t, ti, tj], osems[slot]).wait()

    ivecs = [(iota16 + d0) * 128 for d0 in (0, 16, 32, 48)]

    def transpose(gslot, tslot):
        gb, tb = gbufs[gslot], tbufs[tslot]

        @plsc.parallel_loop(0, IDX_ROW, step=4)
        def _bp(bp0):
            for bj in range(4):
                bp = bp0 + bj
                for k in range(4):
                    vals = gb[bp, pl.ds(k * 16, 16)]
                    plsc.store_scatter(tb, [ivecs[k] + bp], vals)

    for r in range(GB):
        fire_g(r, r)

    @pl.loop(0, N_ROWS, step=GB)
    def _rows(r0):
        for j in range(GB):
            r = r0 + j
            gslot, tslot = j % GB, j % TB
            drain_g(r, gslot)
            if j < TB:
                pl.when(r0 > 0)(lambda rr=r, ts=tslot: wait_outs(rr - TB, ts))
            else:
                wait_outs(r - TB, tslot)
            transpose(gslot, tslot)
            fire_outs(r, tslot)
            pl.when(r0 < N_ROWS - GB)(
                lambda rr=r, gs=gslot: fire_g(rr + GB, gs))

    wait_outs(N_ROWS - 2, 0)
    wait_outs(N_ROWS - 1, 1)


@jax.jit
def _embed(x_flat3, table):
    mesh = plsc.VectorSubcoreMesh(core_axis_name="c", subcore_axis_name="s")
    return pl.kernel(
        _embed_body,
        out_type=jax.ShapeDtypeStruct((50, 8, 128, 1024), jnp.float32),
        mesh=mesh,
        compiler_params=pltpu.CompilerParams(
            use_tc_tiling_on_sc=False, needs_layout_passes=False),
        scratch_types=[
            pltpu.VMEM((N_ROWS, IDX_ROW), jnp.int32),
            pltpu.VMEM((IDX_ROW, D), jnp.float32),
            pltpu.VMEM((IDX_ROW, D), jnp.float32),
            pltpu.VMEM((IDX_ROW, D), jnp.float32),
            pltpu.VMEM((IDX_ROW, D), jnp.float32),
            pltpu.VMEM((8192,), jnp.float32),
            pltpu.VMEM((8192,), jnp.float32),
            pltpu.SemaphoreType.DMA,
            pltpu.SemaphoreType.DMA,
            pltpu.SemaphoreType.DMA,
            pltpu.SemaphoreType.DMA,
            pltpu.SemaphoreType.DMA,
            pltpu.SemaphoreType.DMA,
        ],
    )(x_flat3, table)


def kernel(x, table):
    x_flat3 = x.T.reshape(NW, N_ROWS, IDX_ROW).astype(jnp.int32)
    out5 = _embed(x_flat3, table).reshape(50, 8, 128, 8, 128)
    return out5.transpose(2, 4, 0, 1, 3).reshape(16384, 50, D)


# speedup vs baseline: 1.9645x; 1.3619x over previous
import jax
import jax.numpy as jnp
from jax import lax
from jax.experimental import pallas as pl
from jax.experimental.pallas import tpu as pltpu
from jax.experimental.pallas import tpu_sc as plsc

VOCAB = 1000000
D = 64
B = 16384 * 50

NC = 2
NS = 16
NW = NC * NS
B_PER_W = B // NW
IDX_ROW = 128
N_ROWS = B_PER_W // IDX_ROW
GB = 4
TB = 2


def _row_coords(r_global):
    t = lax.shift_right_logical(r_global, 7)
    tj = lax.bitwise_and(r_global, 127)
    return t, tj


def _embed_body(x_hbm, table_hbm, out_hbm, idx_v,
                g0, g1, g2, g3, t0, t1,
                gs0, gs1, gs2, gs3, os0, os1):
    wid = lax.axis_index("s") * NC + lax.axis_index("c")
    pltpu.sync_copy(x_hbm.at[wid], idx_v)
    row_base = wid * N_ROWS
    gbufs = (g0, g1, g2, g3)
    tbufs = (t0, t1)
    gsems = (gs0, gs1, gs2, gs3)
    osems = (os0, os1)
    iota16 = lax.iota(jnp.int32, 16)

    def fire_g(r, slot):
        pltpu.async_copy(table_hbm.at[idx_v.at[r]], gbufs[slot], gsems[slot])

    def drain_g(r, slot):
        pltpu.make_async_copy(
            table_hbm.at[idx_v.at[r]], gbufs[slot], gsems[slot]).wait()

    def fire_outs(r, slot):
        t, tj = _row_coords(row_base + r)
        for ti in range(8):
            pltpu.async_copy(
                tbufs[slot].at[pl.ds(ti * 1024, 1024)],
                out_hbm.at[t, ti, tj], osems[slot])

    def wait_outs(r, slot):
        t, tj = _row_coords(row_base + r)
        for ti in range(8):
            pltpu.make_async_copy(
                tbufs[slot].at[pl.ds(ti * 1024, 1024)],
                out_hbm.at[t, ti, tj], osems[slot]).wait()

    djs = [jnp.bitwise_and(iota16 + j, 15) for j in range(16)]
    gdst = [djs[j] * 128 + iota16 for j in range(16)]

    def transpose(gslot, tslot):
        gb, tb = gbufs[gslot], tbufs[tslot]

        @plsc.parallel_loop(0, IDX_ROW, step=16)
        def _bp(bp0):
            bpv = iota16 + bp0
            for d0 in (0, 16, 32, 48):
                for j in range(16):
                    vals = plsc.load_gather(gb, [bpv, djs[j] + d0])
                    plsc.store_scatter(tb, [gdst[j] + (d0 * 128 + bp0)], vals)

    for r in range(GB):
        fire_g(r, r)

    @pl.loop(0, N_ROWS, step=GB)
    def _rows(r0):
        for j in range(GB):
            r = r0 + j
            gslot, tslot = j % GB, j % TB
            drain_g(r, gslot)
            if j < TB:
                pl.when(r0 > 0)(lambda rr=r, ts=tslot: wait_outs(rr - TB, ts))
            else:
                wait_outs(r - TB, tslot)
            transpose(gslot, tslot)
            fire_outs(r, tslot)
            pl.when(r0 < N_ROWS - GB)(
                lambda rr=r, gs=gslot: fire_g(rr + GB, gs))

    wait_outs(N_ROWS - 2, 0)
    wait_outs(N_ROWS - 1, 1)


@jax.jit
def _embed(x_flat3, table):
    mesh = plsc.VectorSubcoreMesh(core_axis_name="c", subcore_axis_name="s")
    return pl.kernel(
        _embed_body,
        out_type=jax.ShapeDtypeStruct((50, 8, 128, 1024), jnp.float32),
        mesh=mesh,
        compiler_params=pltpu.CompilerParams(
            use_tc_tiling_on_sc=False, needs_layout_passes=False),
        scratch_types=[
            pltpu.VMEM((N_ROWS, IDX_ROW), jnp.int32),
            pltpu.VMEM((IDX_ROW, D), jnp.float32),
            pltpu.VMEM((IDX_ROW, D), jnp.float32),
            pltpu.VMEM((IDX_ROW, D), jnp.float32),
            pltpu.VMEM((IDX_ROW, D), jnp.float32),
            pltpu.VMEM((8192,), jnp.float32),
            pltpu.VMEM((8192,), jnp.float32),
            pltpu.SemaphoreType.DMA,
            pltpu.SemaphoreType.DMA,
            pltpu.SemaphoreType.DMA,
            pltpu.SemaphoreType.DMA,
            pltpu.SemaphoreType.DMA,
            pltpu.SemaphoreType.DMA,
        ],
    )(x_flat3, table)


def kernel(x, table):
    x_flat3 = x.T.reshape(NW, N_ROWS, IDX_ROW).astype(jnp.int32)
    out5 = _embed(x_flat3, table).reshape(50, 8, 128, 8, 128)
    return out5.transpose(2, 4, 0, 1, 3).reshape(16384, 50, D)
